# baseline (speedup 1.0000x reference)
import jax
import jax.numpy as jnp
from jax.experimental import pallas as pl
from jax.experimental.pallas import tpu as pltpu

_BB = 16


def _mm_kernel(params_ref, x_ref, tx_ref, memt_ref, out_t_ref, out_ref):
    i = pl.program_id(0)
    inv_t = 1.0 / params_ref[0]

    @pl.when(i >= 13)
    def _compute():
        xx = jnp.concatenate([x_ref[...], tx_ref[...]], axis=0) * inv_t
        M = out_ref.shape[1]
        CH = 8192
        for j in range(13):
            w = min(CH, M - j * CH)
            y = jax.lax.dot_general(
                xx, memt_ref[:, j * CH:j * CH + w], (((1,), (0,)), ((), ())),
                preferred_element_type=jnp.float32)
            out_ref[:, j * CH:j * CH + w] = y[:_BB]
            out_t_ref[:, j * CH:j * CH + w] = y[_BB:]


@jax.jit
def kernel(image_features, transformed_image_features, indices, memory, params):
    del indices
    B, D = image_features.shape
    M = memory.shape[0]
    mem_t = memory.T
    grid = (13 + B // _BB,)
    out_shape = jax.ShapeDtypeStruct((B, M), jnp.float32)
    out_t, out = pl.pallas_call(
        _mm_kernel,
        grid=grid,
        in_specs=[
            pl.BlockSpec(memory_space=pltpu.SMEM),
            pl.BlockSpec((_BB, D), lambda i: (jnp.maximum(i - 13, 0), 0)),
            pl.BlockSpec((_BB, D), lambda i: (jnp.maximum(i - 13, 0), 0)),
            pl.BlockSpec((D, M), lambda i: (0, 0)),
        ],
        out_specs=[
            pl.BlockSpec((_BB, M), lambda i: (jnp.maximum(i - 13, 0), 0)),
            pl.BlockSpec((_BB, M), lambda i: (jnp.maximum(i - 13, 0), 0)),
        ],
        out_shape=[out_shape, out_shape],
        compiler_params=pltpu.CompilerParams(
            dimension_semantics=("arbitrary",),
        ),
    )(params, image_features, transformed_image_features, mem_t)
    return (out_t, out)
